# Initial kernel scaffold; baseline (speedup 1.0000x reference)
import jax
import jax.numpy as jnp
from jax.experimental import pallas as pl


def kernel(input_ids, table, prompt_embeddings):
    raise NotImplementedError("write your pallas kernel here")


# speedup vs baseline: 1.2031x; 1.2031x over previous
import functools

import jax
import jax.numpy as jnp
from jax import lax
from jax.experimental import pallas as pl
from jax.experimental.pallas import tpu as pltpu
from jax.experimental.pallas import tpu_sc as plsc

VOCAB = 1000000
DIM = 64
BATCH = 1024
SEQ = 200
N_PROMPT = 20

_info = plsc.get_sparse_core_info()
NC, NS = _info.num_cores, _info.num_subcores
NW = NC * NS
SEQ_PER_W = BATCH // NW
HALF = SEQ // 2

_mesh = plsc.VectorSubcoreMesh(core_axis_name="c", subcore_axis_name="s")


@functools.partial(
    pl.kernel,
    mesh=_mesh,
    out_type=jax.ShapeDtypeStruct((BATCH, SEQ, DIM), jnp.float32),
    compiler_params=pltpu.CompilerParams(use_tc_tiling_on_sc=False),
    scratch_types=[
        pltpu.VMEM((2, HALF), jnp.int32),
        pltpu.VMEM((2, HALF, DIM), jnp.float32),
        pltpu.VMEM((N_PROMPT, DIM), jnp.float32),
        pltpu.SemaphoreType.DMA,
    ],
)
def _emb_lookup(ids_hbm, table_hbm, prompt_hbm, out_hbm,
                idx_v, rows_v, prompt_v, sem):
    wid = lax.axis_index("s") * NC + lax.axis_index("c")
    base = wid * SEQ_PER_W
    pltpu.sync_copy(prompt_hbm, prompt_v)

    def seq_body(i, carry):
        b = base + i
        pltpu.sync_copy(ids_hbm.at[b], idx_v)
        c0 = pltpu.async_copy(table_hbm.at[idx_v.at[0]], rows_v.at[0], sem)
        c1 = pltpu.async_copy(table_hbm.at[idx_v.at[1]], rows_v.at[1], sem)
        pltpu.sync_copy(prompt_v, out_hbm.at[b, pl.ds(0, N_PROMPT)])
        c0.wait()
        c1.wait()
        pltpu.sync_copy(rows_v.at[0, pl.ds(N_PROMPT, HALF - N_PROMPT)],
                        out_hbm.at[b, pl.ds(N_PROMPT, HALF - N_PROMPT)])
        pltpu.sync_copy(rows_v.at[1], out_hbm.at[b, pl.ds(HALF, HALF)])
        return carry

    lax.fori_loop(0, SEQ_PER_W, seq_body, 0)


def kernel(input_ids, table, prompt_embeddings):
    ids3 = input_ids.reshape(BATCH, 2, HALF)
    return _emb_lookup(ids3, table, prompt_embeddings)


# speedup vs baseline: 1.9177x; 1.5939x over previous
import functools

import jax
import jax.numpy as jnp
from jax import lax
from jax.experimental import pallas as pl
from jax.experimental.pallas import tpu as pltpu
from jax.experimental.pallas import tpu_sc as plsc

VOCAB = 1000000
DIM = 64
BATCH = 1024
SEQ = 200
N_PROMPT = 20
REST = SEQ - N_PROMPT
CA = 96
CB = 88
IDSW = CA + CB

_info = plsc.get_sparse_core_info()
NC, NS = _info.num_cores, _info.num_subcores
NW = NC * NS
SEQ_PER_W = BATCH // NW
S = 4
NCHUNK = SEQ_PER_W // S

_mesh = plsc.VectorSubcoreMesh(core_axis_name="c", subcore_axis_name="s")


@functools.partial(
    pl.kernel,
    mesh=_mesh,
    out_type=jax.ShapeDtypeStruct((BATCH * SEQ, DIM), jnp.float32),
    compiler_params=pltpu.CompilerParams(use_tc_tiling_on_sc=False),
    scratch_types=[
        pltpu.VMEM((S, IDSW), jnp.int32),
        pltpu.VMEM((S, IDSW), jnp.int32),
        pltpu.VMEM((S * SEQ, DIM), jnp.float32),
        pltpu.VMEM((S * SEQ, DIM), jnp.float32),
        pltpu.VMEM((N_PROMPT, DIM), jnp.float32),
        pltpu.SemaphoreType.DMA,
        pltpu.SemaphoreType.DMA,
    ],
)
def _emb_lookup(ids_hbm, table_hbm, prompt_hbm, out_hbm,
                idx0, idx1, rows0, rows1, prompt_v, sem0, sem1):
    wid = lax.axis_index("s") * NC + lax.axis_index("c")
    seq_base = wid * SEQ_PER_W
    idx_v = (idx0, idx1)
    rows_v = (rows0, rows1)
    sems = (sem0, sem1)

    pltpu.sync_copy(prompt_hbm, prompt_v)
    for buf in rows_v:
        for s in range(S):
            for r in range(N_PROMPT):
                for c in range(DIM // 16):
                    buf[s * SEQ + r, pl.ds(c * 16, 16)] = (
                        prompt_v[r, pl.ds(c * 16, 16)])

    def stage(g, buf):
        b0 = seq_base + g * S
        pltpu.sync_copy(ids_hbm.at[pl.ds(b0, S)], idx_v[buf])
        copies = []
        for s in range(S):
            copies.append(pltpu.async_copy(
                table_hbm.at[idx_v[buf].at[s, pl.ds(0, CA)]],
                rows_v[buf].at[pl.ds(s * SEQ + N_PROMPT, CA)], sems[buf]))
            copies.append(pltpu.async_copy(
                table_hbm.at[idx_v[buf].at[s, pl.ds(CA, CB)]],
                rows_v[buf].at[pl.ds(s * SEQ + SEQ - CB, CB)], sems[buf]))
        return copies

    pending = stage(0, 0)
    for g in range(NCHUNK):
        buf = g % 2
        if g + 1 < NCHUNK:
            nxt = stage(g + 1, 1 - buf)
        for c in pending:
            c.wait()
        pltpu.sync_copy(rows_v[buf],
                        out_hbm.at[pl.ds((seq_base + g * S) * SEQ, S * SEQ)])
        if g + 1 < NCHUNK:
            pending = nxt


def kernel(input_ids, table, prompt_embeddings):
    ids_p = jnp.concatenate(
        [input_ids[:, N_PROMPT:N_PROMPT + CA],
         input_ids[:, SEQ - CB:]], axis=1)
    out2 = _emb_lookup(ids_p, table, prompt_embeddings)
    return out2.reshape(BATCH, SEQ, DIM)


# speedup vs baseline: 2.0235x; 1.0552x over previous
import functools

import jax
import jax.numpy as jnp
from jax import lax
from jax.experimental import pallas as pl
from jax.experimental.pallas import tpu as pltpu
from jax.experimental.pallas import tpu_sc as plsc

VOCAB = 1000000
DIM = 64
PAD_DIM = 128
BATCH = 1024
SEQ = 200
N_PROMPT = 20
REST = SEQ - N_PROMPT
CA = 96
CB = 88
IDSW = CA + CB

_info = plsc.get_sparse_core_info()
NC, NS = _info.num_cores, _info.num_subcores
NW = NC * NS
SEQ_PER_W = BATCH // NW
S = 2
NCHUNK = SEQ_PER_W // S

_mesh = plsc.VectorSubcoreMesh(core_axis_name="c", subcore_axis_name="s")


@functools.partial(
    pl.kernel,
    mesh=_mesh,
    out_type=jax.ShapeDtypeStruct((BATCH * SEQ, DIM), jnp.float32),
    compiler_params=pltpu.CompilerParams(use_tc_tiling_on_sc=False),
    scratch_types=[
        pltpu.VMEM((S, IDSW), jnp.int32),
        pltpu.VMEM((S, IDSW), jnp.int32),
        pltpu.VMEM((S * SEQ, PAD_DIM), jnp.float32),
        pltpu.VMEM((S * SEQ, PAD_DIM), jnp.float32),
        pltpu.VMEM((N_PROMPT, DIM), jnp.float32),
        pltpu.SemaphoreType.DMA,
        pltpu.SemaphoreType.DMA,
    ],
)
def _emb_lookup(ids_hbm, table_hbm, prompt_hbm, out_hbm,
                idx0, idx1, rows0, rows1, prompt_v, sem0, sem1):
    wid = lax.axis_index("s") * NC + lax.axis_index("c")
    seq_base = wid * SEQ_PER_W
    idx_v = (idx0, idx1)
    rows_v = (rows0, rows1)
    sems = (sem0, sem1)

    pltpu.sync_copy(prompt_hbm, prompt_v)
    for buf in rows_v:
        for s in range(S):
            for r in range(N_PROMPT):
                for c in range(DIM // 16):
                    buf[s * SEQ + r, pl.ds(c * 16, 16)] = (
                        prompt_v[r, pl.ds(c * 16, 16)])

    def stage(g, buf):
        b0 = seq_base + g * S
        pltpu.sync_copy(ids_hbm.at[pl.ds(b0, S)], idx_v[buf])
        copies = []
        for s in range(S):
            copies.append(pltpu.async_copy(
                table_hbm.at[idx_v[buf].at[s, pl.ds(0, CA)]],
                rows_v[buf].at[pl.ds(s * SEQ + N_PROMPT, CA)], sems[buf]))
            copies.append(pltpu.async_copy(
                table_hbm.at[idx_v[buf].at[s, pl.ds(CA, CB)]],
                rows_v[buf].at[pl.ds(s * SEQ + SEQ - CB, CB)], sems[buf]))
        return copies

    pending = stage(0, 0)
    for g in range(NCHUNK):
        buf = g % 2
        if g + 1 < NCHUNK:
            nxt = stage(g + 1, 1 - buf)
        for c in pending:
            c.wait()
        pltpu.sync_copy(rows_v[buf].at[:, pl.ds(0, DIM)],
                        out_hbm.at[pl.ds((seq_base + g * S) * SEQ, S * SEQ)])
        if g + 1 < NCHUNK:
            pending = nxt
    return


def kernel(input_ids, table, prompt_embeddings):
    ids_p = jnp.concatenate(
        [input_ids[:, N_PROMPT:N_PROMPT + CA],
         input_ids[:, SEQ - CB:]], axis=1)
    table_p = jnp.pad(table, ((0, 0), (0, PAD_DIM - DIM)))
    out2 = _emb_lookup(ids_p, table_p, prompt_embeddings)
    return out2.reshape(BATCH, SEQ, DIM)


# speedup vs baseline: 2.2212x; 1.0977x over previous
import functools

import jax
import jax.numpy as jnp
from jax import lax
from jax.experimental import pallas as pl
from jax.experimental.pallas import tpu as pltpu
from jax.experimental.pallas import tpu_sc as plsc

VOCAB = 1000000
DIM = 64
PAD_DIM = 128
BATCH = 1024
SEQ = 200
N_PROMPT = 20
REST = SEQ - N_PROMPT
CA = 96
CB = 88
IDSW = CA + CB

_info = plsc.get_sparse_core_info()
NC, NS = _info.num_cores, _info.num_subcores
NW = NC * NS
SEQ_PER_W = BATCH // NW
S = 2
NCHUNK = SEQ_PER_W // S

_mesh = plsc.VectorSubcoreMesh(core_axis_name="c", subcore_axis_name="s")


@functools.partial(
    pl.kernel,
    mesh=_mesh,
    out_type=jax.ShapeDtypeStruct((BATCH * SEQ, PAD_DIM), jnp.float32),
    compiler_params=pltpu.CompilerParams(use_tc_tiling_on_sc=False),
    scratch_types=[
        pltpu.VMEM((S, IDSW), jnp.int32),
        pltpu.VMEM((S, IDSW), jnp.int32),
        pltpu.VMEM((S * SEQ, PAD_DIM), jnp.float32),
        pltpu.VMEM((S * SEQ, PAD_DIM), jnp.float32),
        pltpu.VMEM((N_PROMPT, DIM), jnp.float32),
        pltpu.SemaphoreType.DMA,
        pltpu.SemaphoreType.DMA,
    ],
)
def _emb_lookup(ids_hbm, table_hbm, prompt_hbm, out_hbm,
                idx0, idx1, rows0, rows1, prompt_v, sem0, sem1):
    wid = lax.axis_index("s") * NC + lax.axis_index("c")
    seq_base = wid * SEQ_PER_W
    idx_v = (idx0, idx1)
    rows_v = (rows0, rows1)
    sems = (sem0, sem1)

    pltpu.sync_copy(prompt_hbm, prompt_v)
    for buf in rows_v:
        for s in range(S):
            for r in range(N_PROMPT):
                for c in range(DIM // 16):
                    buf[s * SEQ + r, pl.ds(c * 16, 16)] = (
                        prompt_v[r, pl.ds(c * 16, 16)])

    def stage(g, buf):
        b0 = seq_base + g * S
        pltpu.sync_copy(ids_hbm.at[pl.ds(b0, S)], idx_v[buf])
        copies = []
        for s in range(S):
            copies.append(pltpu.async_copy(
                table_hbm.at[idx_v[buf].at[s, pl.ds(0, CA)]],
                rows_v[buf].at[pl.ds(s * SEQ + N_PROMPT, CA)], sems[buf]))
            copies.append(pltpu.async_copy(
                table_hbm.at[idx_v[buf].at[s, pl.ds(CA, CB)]],
                rows_v[buf].at[pl.ds(s * SEQ + SEQ - CB, CB)], sems[buf]))
        return copies

    pending = stage(0, 0)
    for g in range(NCHUNK):
        buf = g % 2
        if g + 1 < NCHUNK:
            nxt = stage(g + 1, 1 - buf)
        for c in pending:
            c.wait()
        pltpu.sync_copy(rows_v[buf],
                        out_hbm.at[pl.ds((seq_base + g * S) * SEQ, S * SEQ)])
        if g + 1 < NCHUNK:
            pending = nxt
    return


def kernel(input_ids, table, prompt_embeddings):
    ids_p = jnp.concatenate(
        [input_ids[:, N_PROMPT:N_PROMPT + CA],
         input_ids[:, SEQ - CB:]], axis=1)
    table_p = jnp.pad(table, ((0, 0), (0, PAD_DIM - DIM)))
    out2 = _emb_lookup(ids_p, table_p, prompt_embeddings)
    return out2[:, :DIM].reshape(BATCH, SEQ, DIM)
